# Initial kernel scaffold; baseline (speedup 1.0000x reference)
import jax
import jax.numpy as jnp
from jax.experimental import pallas as pl


def kernel(text, adj_index, adj_weight, W):
    raise NotImplementedError("write your pallas kernel here")


# speedup vs baseline: 9.1851x; 9.1851x over previous
import functools

import jax
import jax.numpy as jnp
from jax import lax
from jax.experimental import pallas as pl
from jax.experimental.pallas import tpu as pltpu
from jax.experimental.pallas import tpu_sc as plsc

N = 10000
E = 320000
D = 128
NC = 2
NS = 16
NW = NC * NS
CH = 128
EPT = 10240
NCH = EPT // CH
E_PAD = NW * EPT
ROWS_PT = N // NS
DEG_PT = 640
N_DEG = NS * DEG_PT

_mesh = plsc.VectorSubcoreMesh(core_axis_name="c", subcore_axis_name="s")


@functools.partial(
    pl.kernel,
    mesh=_mesh,
    out_type=jax.ShapeDtypeStruct((NC, NS, DEG_PT), jnp.float32),
    scratch_types=[
        pltpu.VMEM((CH,), jnp.int32),
        pltpu.VMEM((CH,), jnp.float32),
        pltpu.VMEM_SHARED((N_DEG,), jnp.float32),
    ],
)
def _sc_deg(dst_hbm, w_hbm, z_hbm, out_hbm, dst_v, w_v, acc):
    cid = lax.axis_index("c")
    sid = lax.axis_index("s")
    pltpu.sync_copy(z_hbm, acc.at[pl.ds(sid * DEG_PT, DEG_PT)])
    plsc.subcore_barrier()
    base = (sid * NC + cid) * EPT

    def body(t, carry):
        off = base + t * CH
        pltpu.sync_copy(dst_hbm.at[pl.ds(off, CH)], dst_v)
        pltpu.sync_copy(w_hbm.at[pl.ds(off, CH)], w_v)
        pltpu.sync_copy(w_v, acc.at[dst_v], add=True)
        return carry

    lax.fori_loop(0, NCH, body, 0)
    plsc.subcore_barrier()
    pltpu.sync_copy(acc.at[pl.ds(sid * DEG_PT, DEG_PT)], out_hbm.at[cid, sid])


@functools.partial(
    pl.kernel,
    mesh=_mesh,
    out_type=jax.ShapeDtypeStruct((NC, NS, ROWS_PT, D), jnp.float32),
    scratch_types=[
        pltpu.VMEM((CH,), jnp.int32),
        pltpu.VMEM((CH,), jnp.int32),
        pltpu.VMEM((CH,), jnp.float32),
        pltpu.VMEM((CH, D), jnp.float32),
        pltpu.VMEM_SHARED((N, D), jnp.float32),
        pltpu.SemaphoreType.DMA,
    ],
)
def _sc_main(hs_hbm, src_hbm, dst_hbm, w_hbm, z_hbm, out_hbm,
             src_v, dst_v, w_v, rows_v, acc, sem):
    cid = lax.axis_index("c")
    sid = lax.axis_index("s")
    pltpu.sync_copy(z_hbm, acc.at[pl.ds(sid * ROWS_PT, ROWS_PT)])
    plsc.subcore_barrier()
    base = (sid * NC + cid) * EPT

    def chunk(t, carry):
        off = base + t * CH
        pltpu.sync_copy(src_hbm.at[pl.ds(off, CH)], src_v)
        pltpu.sync_copy(dst_hbm.at[pl.ds(off, CH)], dst_v)
        pltpu.sync_copy(w_hbm.at[pl.ds(off, CH)], w_v)
        pltpu.async_copy(hs_hbm.at[src_v], rows_v, sem).wait()

        def scale(g, c2):
            wv16 = w_v[pl.ds(g * 16, 16)]
            for k in range(16):
                s = wv16[k]
                e = g * 16 + k
                for j in range(D // 16):
                    sl = pl.ds(j * 16, 16)
                    rows_v[e, sl] = rows_v[e, sl] * s
            return c2

        lax.fori_loop(0, CH // 16, scale, 0)
        pltpu.sync_copy(rows_v, acc.at[dst_v], add=True)
        return carry

    lax.fori_loop(0, NCH, chunk, 0)
    plsc.subcore_barrier()
    pltpu.sync_copy(acc.at[pl.ds(sid * ROWS_PT, ROWS_PT)], out_hbm.at[cid, sid])


def _tc_k1_body(text_ref, w_ref, degp_ref, hs_ref, g_ref, dinv_ref):
    h = jnp.dot(text_ref[...], w_ref[...], preferred_element_type=jnp.float32)
    deg = degp_ref[0] + degp_ref[1] + 2.0
    dinv = lax.rsqrt(deg)
    hs_ref[...] = h * dinv
    g_ref[...] = (2.0 * dinv * dinv) * h
    dinv_ref[...] = dinv


def _tc_k2_body(p_ref, g_ref, dinv_ref, out_ref):
    s = dinv_ref[...] * (p_ref[0] + p_ref[1]) + g_ref[...]
    out_ref[...] = jnp.where(s >= 0, s, 0.01 * s)


_BR = 1000


def kernel(text, adj_index, adj_weight, W):
    src = adj_index[0]
    dst = adj_index[1]
    pad = E_PAD - E
    zi = jnp.zeros((pad,), jnp.int32)
    srcp = jnp.concatenate([src, zi])
    dstp = jnp.concatenate([dst, zi])
    wp = jnp.concatenate([adj_weight, jnp.zeros((pad,), jnp.float32)])

    degp = _sc_deg(dstp, wp, jnp.zeros((DEG_PT,), jnp.float32))
    degp3 = degp.reshape(NC, N_DEG, 1)

    nblk = N // _BR
    hs, g, dinv = pl.pallas_call(
        _tc_k1_body,
        grid=(nblk,),
        in_specs=[
            pl.BlockSpec((_BR, D), lambda i: (i, 0)),
            pl.BlockSpec((D, D), lambda i: (0, 0)),
            pl.BlockSpec((NC, _BR, 1), lambda i: (0, i, 0)),
        ],
        out_specs=[
            pl.BlockSpec((_BR, D), lambda i: (i, 0)),
            pl.BlockSpec((_BR, D), lambda i: (i, 0)),
            pl.BlockSpec((_BR, 1), lambda i: (i, 0)),
        ],
        out_shape=[
            jax.ShapeDtypeStruct((N, D), jnp.float32),
            jax.ShapeDtypeStruct((N, D), jnp.float32),
            jax.ShapeDtypeStruct((N, 1), jnp.float32),
        ],
    )(text, W, degp3)

    P = _sc_main(hs, srcp, dstp, wp, jnp.zeros((ROWS_PT, D), jnp.float32))
    P2 = P.reshape(NC, N, D)

    out = pl.pallas_call(
        _tc_k2_body,
        grid=(nblk,),
        in_specs=[
            pl.BlockSpec((NC, _BR, D), lambda i: (0, i, 0)),
            pl.BlockSpec((_BR, D), lambda i: (i, 0)),
            pl.BlockSpec((_BR, 1), lambda i: (i, 0)),
        ],
        out_specs=pl.BlockSpec((_BR, D), lambda i: (i, 0)),
        out_shape=jax.ShapeDtypeStruct((N, D), jnp.float32),
    )(P2, g, dinv)
    return out


# speedup vs baseline: 13.6974x; 1.4913x over previous
import functools

import jax
import jax.numpy as jnp
from jax import lax
from jax.experimental import pallas as pl
from jax.experimental.pallas import tpu as pltpu
from jax.experimental.pallas import tpu_sc as plsc

N = 10000
E = 320000
D = 128
NC = 2
NS = 16
NW = NC * NS
CH = 128
EPT = 10240
NCH = EPT // CH
E_PAD = NW * EPT
ROWS_PT = N // NS
DEG_PT = 640
N_DEG = NS * DEG_PT

_mesh = plsc.VectorSubcoreMesh(core_axis_name="c", subcore_axis_name="s")


@functools.partial(
    pl.kernel,
    mesh=_mesh,
    out_type=jax.ShapeDtypeStruct((NC, NS, DEG_PT), jnp.float32),
    scratch_types=[
        pltpu.VMEM((NCH, CH), jnp.int32),
        pltpu.VMEM((NCH, CH), jnp.float32),
        pltpu.VMEM_SHARED((N_DEG,), jnp.float32),
        pltpu.SemaphoreType.DMA,
    ],
)
def _sc_deg(dst_hbm, w_hbm, z_hbm, out_hbm, dst_all, w_all, acc, sem):
    cid = lax.axis_index("c")
    sid = lax.axis_index("s")
    wid = sid * NC + cid
    pltpu.sync_copy(z_hbm, acc.at[pl.ds(sid * DEG_PT, DEG_PT)])
    pltpu.sync_copy(dst_hbm.at[wid], dst_all)
    pltpu.sync_copy(w_hbm.at[wid], w_all)
    plsc.subcore_barrier()

    def body(q, carry):
        for r in range(4):
            t = q * 4 + r
            pltpu.async_copy(w_all.at[t], acc.at[dst_all.at[t]], sem, add=True)
        for r in range(4):
            t = q * 4 + r
            pltpu.make_async_copy(w_all.at[t], acc.at[dst_all.at[t]], sem).wait()
        return carry

    lax.fori_loop(0, NCH // 4, body, 0)
    plsc.subcore_barrier()
    pltpu.sync_copy(acc.at[pl.ds(sid * DEG_PT, DEG_PT)], out_hbm.at[cid, sid])


@functools.partial(
    pl.kernel,
    mesh=_mesh,
    out_type=jax.ShapeDtypeStruct((NC, NS, ROWS_PT, D), jnp.float32),
    scratch_types=[
        pltpu.VMEM((NCH // 2, CH), jnp.int32),
        pltpu.VMEM((NCH // 2, CH), jnp.int32),
        pltpu.VMEM((NCH // 2, CH), jnp.float32),
        pltpu.VMEM((CH, D), jnp.float32),
        pltpu.VMEM((CH, D), jnp.float32),
        pltpu.VMEM_SHARED((N, D), jnp.float32),
        pltpu.SemaphoreType.DMA,
        pltpu.SemaphoreType.DMA,
    ],
)
def _sc_main(hs_hbm, src_hbm, dst_hbm, w_hbm, z_hbm, out_hbm,
             src_all, dst_all, w_all, rows_a, rows_b, acc, sem_a, sem_b):
    cid = lax.axis_index("c")
    sid = lax.axis_index("s")
    wid = sid * NC + cid
    hch = NCH // 2
    pltpu.sync_copy(z_hbm, acc.at[pl.ds(sid * ROWS_PT, ROWS_PT)])
    plsc.subcore_barrier()

    def _half(t, rows, sem, nxt):
        pltpu.make_async_copy(hs_hbm.at[src_all.at[t]], rows, sem).wait()

        def scale(g, c2):
            wv16 = w_all[t, pl.ds(g * 16, 16)]
            for k in range(16):
                s = wv16[k]
                e = g * 16 + k
                for j in range(D // 16):
                    sl = pl.ds(j * 16, 16)
                    rows[e, sl] = rows[e, sl] * s
            return c2

        lax.fori_loop(0, CH // 16, scale, 0)
        pltpu.sync_copy(rows, acc.at[dst_all.at[t]], add=True)

        @pl.when(nxt < hch)
        def _():
            pltpu.async_copy(hs_hbm.at[src_all.at[nxt]], rows, sem)

    def chunk(tt, carry):
        t0 = 2 * tt
        _half(t0, rows_a, sem_a, t0 + 2)
        _half(t0 + 1, rows_b, sem_b, t0 + 3)
        return carry

    for h in range(2):
        pltpu.sync_copy(src_hbm.at[wid, pl.ds(h * hch, hch)], src_all)
        pltpu.sync_copy(dst_hbm.at[wid, pl.ds(h * hch, hch)], dst_all)
        pltpu.sync_copy(w_hbm.at[wid, pl.ds(h * hch, hch)], w_all)
        pltpu.async_copy(hs_hbm.at[src_all.at[0]], rows_a, sem_a)
        pltpu.async_copy(hs_hbm.at[src_all.at[1]], rows_b, sem_b)
        lax.fori_loop(0, hch // 2, chunk, 0)

    plsc.subcore_barrier()
    pltpu.sync_copy(acc.at[pl.ds(sid * ROWS_PT, ROWS_PT)], out_hbm.at[cid, sid])


def _tc_k1_body(text_ref, w_ref, degp_ref, hs_ref, g_ref, dinv_ref):
    h = jnp.dot(text_ref[...], w_ref[...], preferred_element_type=jnp.float32)
    deg = degp_ref[0] + degp_ref[1] + 2.0
    dinv = lax.rsqrt(deg)
    hs_ref[...] = h * dinv
    g_ref[...] = (2.0 * dinv * dinv) * h
    dinv_ref[...] = dinv


def _tc_k2_body(p_ref, g_ref, dinv_ref, out_ref):
    s = dinv_ref[...] * (p_ref[0] + p_ref[1]) + g_ref[...]
    out_ref[...] = jnp.where(s >= 0, s, 0.01 * s)


_BR = 1000


def kernel(text, adj_index, adj_weight, W):
    src = adj_index[0]
    dst = adj_index[1]
    pad = E_PAD - E
    zi = jnp.zeros((pad,), jnp.int32)
    srcp = jnp.concatenate([src, zi]).reshape(NW, NCH, CH)
    dstp = jnp.concatenate([dst, zi]).reshape(NW, NCH, CH)
    wp = jnp.concatenate([adj_weight, jnp.zeros((pad,), jnp.float32)]).reshape(NW, NCH, CH)

    degp = _sc_deg(dstp, wp, jnp.zeros((DEG_PT,), jnp.float32))
    degp3 = degp.reshape(NC, N_DEG, 1)

    nblk = N // _BR
    hs, g, dinv = pl.pallas_call(
        _tc_k1_body,
        grid=(nblk,),
        in_specs=[
            pl.BlockSpec((_BR, D), lambda i: (i, 0)),
            pl.BlockSpec((D, D), lambda i: (0, 0)),
            pl.BlockSpec((NC, _BR, 1), lambda i: (0, i, 0)),
        ],
        out_specs=[
            pl.BlockSpec((_BR, D), lambda i: (i, 0)),
            pl.BlockSpec((_BR, D), lambda i: (i, 0)),
            pl.BlockSpec((_BR, 1), lambda i: (i, 0)),
        ],
        out_shape=[
            jax.ShapeDtypeStruct((N, D), jnp.float32),
            jax.ShapeDtypeStruct((N, D), jnp.float32),
            jax.ShapeDtypeStruct((N, 1), jnp.float32),
        ],
    )(text, W, degp3)

    P = _sc_main(hs, srcp, dstp, wp, jnp.zeros((ROWS_PT, D), jnp.float32))
    P2 = P.reshape(NC, N, D)

    out = pl.pallas_call(
        _tc_k2_body,
        grid=(nblk,),
        in_specs=[
            pl.BlockSpec((NC, _BR, D), lambda i: (0, i, 0)),
            pl.BlockSpec((_BR, D), lambda i: (i, 0)),
            pl.BlockSpec((_BR, 1), lambda i: (i, 0)),
        ],
        out_specs=pl.BlockSpec((_BR, D), lambda i: (i, 0)),
        out_shape=jax.ShapeDtypeStruct((N, D), jnp.float32),
    )(P2, g, dinv)
    return out
